# baseline (speedup 1.0000x reference)
import functools

import jax
import jax.numpy as jnp
from jax import lax
from jax.experimental import pallas as pl
from jax.experimental.pallas import tpu as pltpu
from jax.experimental.pallas import tpu_sc as plsc

N = 10000
E = 320000
D = 128
G = 64
DO = 16
EPS_BN = 1e-5

NC = 2
NS = 16
NW = NC * NS
EPW = E // NW
K = 125
CH = EPW // K
NPAD = 10240
ROWS_PT = NPAD // NS
DPT = NPAD // NS
ZR = 64
SUP = 40
NSB = CH // SUP

R = 1000
NB = N // R

_mesh = plsc.VectorSubcoreMesh(core_axis_name="c", subcore_axis_name="s")


@functools.partial(
    pl.kernel,
    out_type=jax.ShapeDtypeStruct((NC * NPAD,), jnp.float32),
    mesh=_mesh,
    scratch_types=[
        pltpu.VMEM_SHARED((NPAD,), jnp.float32),
        pltpu.VMEM((CH, K), jnp.int32),
        pltpu.VMEM((K,), jnp.float32),
        pltpu.VMEM((DPT,), jnp.float32),
    ],
)
def _deg_kernel(dst_hbm, out_hbm, deg_sp, idx_v, ones_v, z_v):
    cid = lax.axis_index("c")
    sid = lax.axis_index("s")
    wid = sid * NC + cid
    zero16 = jnp.zeros((16,), jnp.float32)
    one16 = jnp.ones((16,), jnp.float32)

    def _zb(i, c):
        z_v[pl.ds(i * 16, 16)] = zero16
        return c

    lax.fori_loop(0, DPT // 16, _zb, 0)
    for k in range(8):
        ones_v[pl.ds(min(k * 16, K - 16), 16)] = one16

    pltpu.sync_copy(z_v, deg_sp.at[pl.ds(sid * DPT, DPT)])
    plsc.subcore_barrier()

    pltpu.sync_copy(dst_hbm.at[wid], idx_v)

    def _body(j, c):
        pltpu.sync_copy(ones_v, deg_sp.at[idx_v.at[j]], add=True)
        return c

    lax.fori_loop(0, CH, _body, 0)
    plsc.subcore_barrier()
    pltpu.sync_copy(deg_sp.at[pl.ds(sid * DPT, DPT)],
                    out_hbm.at[pl.ds(cid * NPAD + sid * DPT, DPT)])


@functools.partial(
    pl.kernel,
    out_type=jax.ShapeDtypeStruct((NC, NPAD, D), jnp.float32),
    mesh=_mesh,
    scratch_types=[
        pltpu.VMEM_SHARED((NPAD, D), jnp.float32),
        pltpu.VMEM((SUP, K), jnp.int32),
        pltpu.VMEM((SUP, K), jnp.int32),
        pltpu.VMEM((2, K, D), jnp.float32),
        pltpu.SemaphoreType.DMA,
        pltpu.SemaphoreType.DMA,
        pltpu.SemaphoreType.DMA,
        pltpu.SemaphoreType.DMA,
    ],
)
def _edge_kernel(y_hbm, src_hbm, dst_hbm, out_hbm,
                 acc_sp, src_v, dst_v, rows_v, sem0, sem1, sems0, sems1):
    cid = lax.axis_index("c")
    sid = lax.axis_index("s")
    wid = sid * NC + cid
    zero16 = jnp.zeros((16,), jnp.float32)

    def _zrow(i, c):
        for k in range(D // 16):
            rows_v[0, i, pl.ds(k * 16, 16)] = zero16
        return c

    lax.fori_loop(0, ZR, _zrow, 0)
    base = sid * ROWS_PT
    for t in range(ROWS_PT // ZR):
        pltpu.sync_copy(rows_v.at[0, pl.ds(0, ZR)],
                        acc_sp.at[pl.ds(base + t * ZR, ZR)])
    plsc.subcore_barrier()

    def _sb_body(sb, c):
        pltpu.sync_copy(src_hbm.at[wid, pl.ds(sb * SUP, SUP)], src_v)
        pltpu.sync_copy(dst_hbm.at[wid, pl.ds(sb * SUP, SUP)], dst_v)
        pltpu.async_copy(y_hbm.at[src_v.at[0]], rows_v.at[0], sem0)
        pltpu.async_copy(y_hbm.at[src_v.at[1]], rows_v.at[1], sem1)

        def _body(g, c2):
            j = 2 * g
            pltpu.make_async_copy(y_hbm.at[src_v.at[j]], rows_v.at[0], sem0).wait()
            s0 = pltpu.async_copy(rows_v.at[0], acc_sp.at[dst_v.at[j]], sems0,
                                  add=True)
            pltpu.make_async_copy(y_hbm.at[src_v.at[j + 1]], rows_v.at[1],
                                  sem1).wait()
            s1 = pltpu.async_copy(rows_v.at[1], acc_sp.at[dst_v.at[j + 1]], sems1,
                                  add=True)
            s0.wait()

            @pl.when(j + 2 < SUP)
            def _():
                pltpu.async_copy(y_hbm.at[src_v.at[j + 2]], rows_v.at[0], sem0)

            s1.wait()

            @pl.when(j + 3 < SUP)
            def _():
                pltpu.async_copy(y_hbm.at[src_v.at[j + 3]], rows_v.at[1], sem1)

            return c2

        lax.fori_loop(0, SUP // 2, _body, 0)
        return c

    lax.fori_loop(0, NSB, _sb_body, 0)
    plsc.subcore_barrier()
    pltpu.sync_copy(acc_sp.at[pl.ds(base, ROWS_PT)],
                    out_hbm.at[cid, pl.ds(base, ROWS_PT)])


def _prep_body(deg0_ref, deg1_ref, x_ref, w_ref, dinv_ref, y_ref):
    dinv = lax.rsqrt(deg0_ref[...] + deg1_ref[...] + 1.0)
    dinv_ref[...] = dinv
    y_ref[...] = jnp.dot(x_ref[...], w_ref[...],
                         preferred_element_type=jnp.float32) * dinv


_prep_call = pl.pallas_call(
    _prep_body,
    grid=(NB,),
    in_specs=[
        pl.BlockSpec((R, 1), lambda i: (i, 0)),
        pl.BlockSpec((R, 1), lambda i: (i, 0)),
        pl.BlockSpec((R, D), lambda i: (i, 0)),
        pl.BlockSpec((D, D), lambda i: (0, 0)),
    ],
    out_specs=[
        pl.BlockSpec((R, 1), lambda i: (i, 0)),
        pl.BlockSpec((R, D), lambda i: (i, 0)),
    ],
    out_shape=[
        jax.ShapeDtypeStruct((N, 1), jnp.float32),
        jax.ShapeDtypeStruct((N, D), jnp.float32),
    ],
)


def _bn_h(z, stats_ref, g_ref, bb_ref):
    mu = stats_ref[0:1] * (1.0 / N)
    ms = stats_ref[1:2] * (1.0 / N)
    var = ms - mu * mu
    inv = lax.rsqrt(var + EPS_BN)
    return jnp.maximum((z - mu) * inv * g_ref[...] + bb_ref[...], 0.0)


def _stats_update(stats_ref, z, i):
    s = jnp.sum(z, axis=0, keepdims=True)
    ss = jnp.sum(z * z, axis=0, keepdims=True)
    upd = jnp.concatenate([s, ss, jnp.zeros((6, D), jnp.float32)], axis=0)

    @pl.when(i == 0)
    def _():
        stats_ref[...] = jnp.zeros((8, D), jnp.float32)

    stats_ref[...] += upd


def _layer_body(acc_ref, y_ref, dinv_ref, b_ref, g_ref, bb_ref, w_ref,
                out_ref, zbuf, stats_ref):
    p = pl.program_id(0)
    i = pl.program_id(1)

    @pl.when(p == 0)
    def _():
        z = (acc_ref[0] + acc_ref[1] + y_ref[...]) * dinv_ref[...] + b_ref[...]
        zbuf[pl.ds(i * R, R), :] = z
        _stats_update(stats_ref, z, i)

    @pl.when(p == 1)
    def _():
        h = _bn_h(zbuf[pl.ds(i * R, R), :], stats_ref, g_ref, bb_ref)
        out_ref[...] = jnp.dot(h, w_ref[...],
                               preferred_element_type=jnp.float32) * dinv_ref[...]


_layer_call = pl.pallas_call(
    _layer_body,
    grid=(2, NB),
    in_specs=[
        pl.BlockSpec((2, R, D), lambda p, i: (0, i * (1 - p), 0)),
        pl.BlockSpec((R, D), lambda p, i: (i * (1 - p), 0)),
        pl.BlockSpec((R, 1), lambda p, i: (i, 0)),
        pl.BlockSpec((1, D), lambda p, i: (0, 0)),
        pl.BlockSpec((1, D), lambda p, i: (0, 0)),
        pl.BlockSpec((1, D), lambda p, i: (0, 0)),
        pl.BlockSpec((D, D), lambda p, i: (0, 0)),
    ],
    out_specs=[pl.BlockSpec((R, D), lambda p, i: (i * p, 0))],
    out_shape=[jax.ShapeDtypeStruct((N, D), jnp.float32)],
    scratch_shapes=[
        pltpu.VMEM((N, D), jnp.float32),
        pltpu.VMEM((8, D), jnp.float32),
    ],
)


def _layer3_body(acc_ref, y_ref, dinv_ref, b_ref, g_ref, bb_ref,
                 gw_ref, gb_ref, batch_ref, fcw_ref, fcb_ref,
                 out_ref, zbuf, logit_s, m_s, s_acc, u_acc, stats_ref):
    p = pl.program_id(0)
    i = pl.program_id(1)

    @pl.when(p == 0)
    def _():
        z = (acc_ref[0] + acc_ref[1] + y_ref[...]) * dinv_ref[...] + b_ref[...]
        zbuf[pl.ds(i * R, R), :] = z
        _stats_update(stats_ref, z, i)

    @pl.when(p == 1)
    def _():
        h = _bn_h(zbuf[pl.ds(i * R, R), :], stats_ref, g_ref, bb_ref)
        zbuf[pl.ds(i * R, R), :] = h
        logit = jnp.dot(h, gw_ref[...],
                        preferred_element_type=jnp.float32) + gb_ref[...]
        logit_s[pl.ds(i * R, R), :] = logit
        seg = lax.broadcasted_iota(jnp.int32, (R, G), 1)
        mask = seg == batch_ref[...]
        part = jnp.max(jnp.where(mask, logit, -1e30), axis=0, keepdims=True)

        @pl.when(i == 0)
        def _():
            m_s[...] = jnp.full((1, G), -1e30, jnp.float32)

        m_s[...] = jnp.maximum(m_s[...], part)

    @pl.when(p == 2)
    def _():
        h = zbuf[pl.ds(i * R, R), :]
        logit = logit_s[pl.ds(i * R, R), :]
        seg = lax.broadcasted_iota(jnp.int32, (R, G), 1)
        mask = (seg == batch_ref[...]).astype(jnp.float32)
        mnode = lax.dot_general(mask, m_s[...], (((1,), (1,)), ((), ())),
                                preferred_element_type=jnp.float32)
        ex = jnp.exp(logit - mnode)
        spart = lax.dot_general(mask, ex, (((0,), (0,)), ((), ())),
                                preferred_element_type=jnp.float32)
        upart = lax.dot_general(mask, h * ex, (((0,), (0,)), ((), ())),
                                preferred_element_type=jnp.float32)

        @pl.when(i == 0)
        def _():
            s_acc[...] = jnp.zeros((G, 1), jnp.float32)
            u_acc[...] = jnp.zeros((G, D), jnp.float32)

        s_acc[...] += spart
        u_acc[...] += upart

        @pl.when(i == NB - 1)
        def _():
            pooled = u_acc[...] / (s_acc[...] + 1e-16)
            out_ref[...] = jnp.dot(pooled, fcw_ref[...],
                                   preferred_element_type=jnp.float32) + fcb_ref[...]


_layer3_call = pl.pallas_call(
    _layer3_body,
    grid=(3, NB),
    in_specs=[
        pl.BlockSpec((2, R, D), lambda p, i: (0, jnp.where(p == 0, i, 0), 0)),
        pl.BlockSpec((R, D), lambda p, i: (jnp.where(p == 0, i, 0), 0)),
        pl.BlockSpec((R, 1), lambda p, i: (i, 0)),
        pl.BlockSpec((1, D), lambda p, i: (0, 0)),
        pl.BlockSpec((1, D), lambda p, i: (0, 0)),
        pl.BlockSpec((1, D), lambda p, i: (0, 0)),
        pl.BlockSpec((D, 1), lambda p, i: (0, 0)),
        pl.BlockSpec((1, 1), lambda p, i: (0, 0)),
        pl.BlockSpec((R, 1), lambda p, i: (i, 0)),
        pl.BlockSpec((D, DO), lambda p, i: (0, 0)),
        pl.BlockSpec((1, DO), lambda p, i: (0, 0)),
    ],
    out_specs=[pl.BlockSpec((G, DO), lambda p, i: (0, 0))],
    out_shape=[jax.ShapeDtypeStruct((G, DO), jnp.float32)],
    scratch_shapes=[
        pltpu.VMEM((N, D), jnp.float32),
        pltpu.VMEM((N, 1), jnp.float32),
        pltpu.VMEM((1, G), jnp.float32),
        pltpu.VMEM((G, 1), jnp.float32),
        pltpu.VMEM((G, D), jnp.float32),
        pltpu.VMEM((8, D), jnp.float32),
    ],
)


def kernel(x, edge_index, batch, W1, b1, W2, b2, W3, b3,
           bn1_g, bn1_b, bn2_g, bn2_b, bn3_g, bn3_b,
           gate_W, gate_b, fc_W, fc_b):
    src = edge_index[0].reshape(NW, CH, K)
    dst = edge_index[1].reshape(NW, CH, K)
    batchc = batch.reshape(N, 1)

    deg2 = _deg_kernel(dst)
    deg0 = deg2[:N].reshape(N, 1)
    deg1 = deg2[NPAD:NPAD + N].reshape(N, 1)
    dinv, y = _prep_call(deg0, deg1, x, W1)

    acc = _edge_kernel(y, src, dst)
    y = _layer_call(acc, y, dinv, b1.reshape(1, D),
                    bn1_g.reshape(1, D), bn1_b.reshape(1, D), W2)[0]

    acc = _edge_kernel(y, src, dst)
    y = _layer_call(acc, y, dinv, b2.reshape(1, D),
                    bn2_g.reshape(1, D), bn2_b.reshape(1, D), W3)[0]

    acc = _edge_kernel(y, src, dst)
    out = _layer3_call(acc, y, dinv, b3.reshape(1, D),
                       bn3_g.reshape(1, D), bn3_b.reshape(1, D),
                       gate_W, gate_b.reshape(1, 1), batchc,
                       fc_W, fc_b.reshape(1, DO))[0]
    return out


# speedup vs baseline: 1.2306x; 1.2306x over previous
import functools

import jax
import jax.numpy as jnp
from jax import lax
from jax.experimental import pallas as pl
from jax.experimental.pallas import tpu as pltpu
from jax.experimental.pallas import tpu_sc as plsc

N = 10000
E = 320000
D = 128
G = 64
DO = 16
EPS_BN = 1e-5

NC = 2
NS = 16
NW = NC * NS
EPW = E // NW
K = 125
CH = EPW // K
NPAD = 10240
ROWS_PT = NPAD // NS
DPT = NPAD // NS
ZR = 64
SUP = 40
NSB = CH // SUP

R = 1000
NB = N // R

_mesh = plsc.VectorSubcoreMesh(core_axis_name="c", subcore_axis_name="s")


@functools.partial(
    pl.kernel,
    out_type=jax.ShapeDtypeStruct((NC * NPAD,), jnp.float32),
    mesh=_mesh,
    scratch_types=[
        pltpu.VMEM_SHARED((NPAD,), jnp.float32),
        pltpu.VMEM((CH, K), jnp.int32),
        pltpu.VMEM((K,), jnp.float32),
        pltpu.VMEM((DPT,), jnp.float32),
    ],
)
def _deg_kernel(dst_hbm, out_hbm, deg_sp, idx_v, ones_v, z_v):
    cid = lax.axis_index("c")
    sid = lax.axis_index("s")
    wid = sid * NC + cid
    zero16 = jnp.zeros((16,), jnp.float32)
    one16 = jnp.ones((16,), jnp.float32)

    def _zb(i, c):
        z_v[pl.ds(i * 16, 16)] = zero16
        return c

    lax.fori_loop(0, DPT // 16, _zb, 0)
    for k in range(8):
        ones_v[pl.ds(min(k * 16, K - 16), 16)] = one16

    pltpu.sync_copy(z_v, deg_sp.at[pl.ds(sid * DPT, DPT)])
    plsc.subcore_barrier()

    pltpu.sync_copy(dst_hbm.at[wid], idx_v)

    def _body(j, c):
        pltpu.sync_copy(ones_v, deg_sp.at[idx_v.at[j]], add=True)
        return c

    lax.fori_loop(0, CH, _body, 0)
    plsc.subcore_barrier()
    pltpu.sync_copy(deg_sp.at[pl.ds(sid * DPT, DPT)],
                    out_hbm.at[pl.ds(cid * NPAD + sid * DPT, DPT)])


@functools.partial(
    pl.kernel,
    out_type=jax.ShapeDtypeStruct((NC, NPAD, D), jnp.float32),
    mesh=_mesh,
    scratch_types=[
        pltpu.VMEM_SHARED((NPAD, D), jnp.float32),
        pltpu.VMEM((SUP, K), jnp.int32),
        pltpu.VMEM((SUP, K), jnp.int32),
        pltpu.VMEM((2, K, D), jnp.float32),
        pltpu.SemaphoreType.DMA,
        pltpu.SemaphoreType.DMA,
    ],
)
def _edge_kernel(y_hbm, src_hbm, dst_hbm, out_hbm,
                 acc_sp, src_v, dst_v, rows_v, sem0, sem1):
    cid = lax.axis_index("c")
    sid = lax.axis_index("s")
    wid = sid * NC + cid
    zero16 = jnp.zeros((16,), jnp.float32)

    def _zrow(i, c):
        for k in range(D // 16):
            rows_v[0, i, pl.ds(k * 16, 16)] = zero16
        return c

    lax.fori_loop(0, ZR, _zrow, 0)
    base = sid * ROWS_PT
    for t in range(ROWS_PT // ZR):
        pltpu.sync_copy(rows_v.at[0, pl.ds(0, ZR)],
                        acc_sp.at[pl.ds(base + t * ZR, ZR)])
    plsc.subcore_barrier()

    def _sb_body(sb, c):
        pltpu.sync_copy(src_hbm.at[wid, pl.ds(sb * SUP, SUP)], src_v)
        pltpu.sync_copy(dst_hbm.at[wid, pl.ds(sb * SUP, SUP)], dst_v)
        pltpu.async_copy(y_hbm.at[src_v.at[0]], rows_v.at[0], sem0)

        def _body(g, c2):
            j = 2 * g
            cp1 = pltpu.async_copy(y_hbm.at[src_v.at[j + 1]], rows_v.at[1], sem1)
            pltpu.make_async_copy(y_hbm.at[src_v.at[j]], rows_v.at[0], sem0).wait()
            pltpu.sync_copy(rows_v.at[0], acc_sp.at[dst_v.at[j]], add=True)

            @pl.when(j + 2 < SUP)
            def _():
                pltpu.async_copy(y_hbm.at[src_v.at[j + 2]], rows_v.at[0], sem0)

            cp1.wait()
            pltpu.sync_copy(rows_v.at[1], acc_sp.at[dst_v.at[j + 1]], add=True)
            return c2

        lax.fori_loop(0, SUP // 2, _body, 0)
        return c

    lax.fori_loop(0, NSB, _sb_body, 0)
    plsc.subcore_barrier()
    pltpu.sync_copy(acc_sp.at[pl.ds(base, ROWS_PT)],
                    out_hbm.at[cid, pl.ds(base, ROWS_PT)])


def _prep_body(deg0_ref, deg1_ref, x_ref, w_ref, dinv_ref, y_ref):
    dinv = lax.rsqrt(deg0_ref[...] + deg1_ref[...] + 1.0)
    dinv_ref[...] = dinv
    y_ref[...] = jnp.dot(x_ref[...], w_ref[...],
                         preferred_element_type=jnp.float32) * dinv


_prep_call = pl.pallas_call(
    _prep_body,
    grid=(NB,),
    in_specs=[
        pl.BlockSpec((R, 1), lambda i: (i, 0)),
        pl.BlockSpec((R, 1), lambda i: (i, 0)),
        pl.BlockSpec((R, D), lambda i: (i, 0)),
        pl.BlockSpec((D, D), lambda i: (0, 0)),
    ],
    out_specs=[
        pl.BlockSpec((R, 1), lambda i: (i, 0)),
        pl.BlockSpec((R, D), lambda i: (i, 0)),
    ],
    out_shape=[
        jax.ShapeDtypeStruct((N, 1), jnp.float32),
        jax.ShapeDtypeStruct((N, D), jnp.float32),
    ],
)


def _bn_h(z, stats_ref, g_ref, bb_ref):
    mu = stats_ref[0:1] * (1.0 / N)
    ms = stats_ref[1:2] * (1.0 / N)
    var = ms - mu * mu
    inv = lax.rsqrt(var + EPS_BN)
    return jnp.maximum((z - mu) * inv * g_ref[...] + bb_ref[...], 0.0)


def _stats_update(stats_ref, z, i):
    s = jnp.sum(z, axis=0, keepdims=True)
    ss = jnp.sum(z * z, axis=0, keepdims=True)
    upd = jnp.concatenate([s, ss, jnp.zeros((6, D), jnp.float32)], axis=0)

    @pl.when(i == 0)
    def _():
        stats_ref[...] = jnp.zeros((8, D), jnp.float32)

    stats_ref[...] += upd


def _layer_body(acc_ref, y_ref, dinv_ref, b_ref, g_ref, bb_ref, w_ref,
                out_ref, zbuf, stats_ref):
    p = pl.program_id(0)
    i = pl.program_id(1)

    @pl.when(p == 0)
    def _():
        z = (acc_ref[0] + acc_ref[1] + y_ref[...]) * dinv_ref[...] + b_ref[...]
        zbuf[pl.ds(i * R, R), :] = z
        _stats_update(stats_ref, z, i)

    @pl.when(p == 1)
    def _():
        h = _bn_h(zbuf[pl.ds(i * R, R), :], stats_ref, g_ref, bb_ref)
        out_ref[...] = jnp.dot(h, w_ref[...],
                               preferred_element_type=jnp.float32) * dinv_ref[...]


_layer_call = pl.pallas_call(
    _layer_body,
    grid=(2, NB),
    in_specs=[
        pl.BlockSpec((2, R, D), lambda p, i: (0, i * (1 - p), 0)),
        pl.BlockSpec((R, D), lambda p, i: (i * (1 - p), 0)),
        pl.BlockSpec((R, 1), lambda p, i: (i, 0)),
        pl.BlockSpec((1, D), lambda p, i: (0, 0)),
        pl.BlockSpec((1, D), lambda p, i: (0, 0)),
        pl.BlockSpec((1, D), lambda p, i: (0, 0)),
        pl.BlockSpec((D, D), lambda p, i: (0, 0)),
    ],
    out_specs=[pl.BlockSpec((R, D), lambda p, i: (i * p, 0))],
    out_shape=[jax.ShapeDtypeStruct((N, D), jnp.float32)],
    scratch_shapes=[
        pltpu.VMEM((N, D), jnp.float32),
        pltpu.VMEM((8, D), jnp.float32),
    ],
)


def _layer3_body(acc_ref, y_ref, dinv_ref, b_ref, g_ref, bb_ref,
                 gw_ref, gb_ref, batch_ref, fcw_ref, fcb_ref,
                 out_ref, zbuf, logit_s, m_s, s_acc, u_acc, stats_ref):
    p = pl.program_id(0)
    i = pl.program_id(1)

    @pl.when(p == 0)
    def _():
        z = (acc_ref[0] + acc_ref[1] + y_ref[...]) * dinv_ref[...] + b_ref[...]
        zbuf[pl.ds(i * R, R), :] = z
        _stats_update(stats_ref, z, i)

    @pl.when(p == 1)
    def _():
        h = _bn_h(zbuf[pl.ds(i * R, R), :], stats_ref, g_ref, bb_ref)
        zbuf[pl.ds(i * R, R), :] = h
        logit = jnp.dot(h, gw_ref[...],
                        preferred_element_type=jnp.float32) + gb_ref[...]
        logit_s[pl.ds(i * R, R), :] = logit
        seg = lax.broadcasted_iota(jnp.int32, (R, G), 1)
        mask = seg == batch_ref[...]
        part = jnp.max(jnp.where(mask, logit, -1e30), axis=0, keepdims=True)

        @pl.when(i == 0)
        def _():
            m_s[...] = jnp.full((1, G), -1e30, jnp.float32)

        m_s[...] = jnp.maximum(m_s[...], part)

    @pl.when(p == 2)
    def _():
        h = zbuf[pl.ds(i * R, R), :]
        logit = logit_s[pl.ds(i * R, R), :]
        seg = lax.broadcasted_iota(jnp.int32, (R, G), 1)
        mask = (seg == batch_ref[...]).astype(jnp.float32)
        mnode = lax.dot_general(mask, m_s[...], (((1,), (1,)), ((), ())),
                                preferred_element_type=jnp.float32)
        ex = jnp.exp(logit - mnode)
        spart = lax.dot_general(mask, ex, (((0,), (0,)), ((), ())),
                                preferred_element_type=jnp.float32)
        upart = lax.dot_general(mask, h * ex, (((0,), (0,)), ((), ())),
                                preferred_element_type=jnp.float32)

        @pl.when(i == 0)
        def _():
            s_acc[...] = jnp.zeros((G, 1), jnp.float32)
            u_acc[...] = jnp.zeros((G, D), jnp.float32)

        s_acc[...] += spart
        u_acc[...] += upart

        @pl.when(i == NB - 1)
        def _():
            pooled = u_acc[...] / (s_acc[...] + 1e-16)
            out_ref[...] = jnp.dot(pooled, fcw_ref[...],
                                   preferred_element_type=jnp.float32) + fcb_ref[...]


_layer3_call = pl.pallas_call(
    _layer3_body,
    grid=(3, NB),
    in_specs=[
        pl.BlockSpec((2, R, D), lambda p, i: (0, jnp.where(p == 0, i, 0), 0)),
        pl.BlockSpec((R, D), lambda p, i: (jnp.where(p == 0, i, 0), 0)),
        pl.BlockSpec((R, 1), lambda p, i: (i, 0)),
        pl.BlockSpec((1, D), lambda p, i: (0, 0)),
        pl.BlockSpec((1, D), lambda p, i: (0, 0)),
        pl.BlockSpec((1, D), lambda p, i: (0, 0)),
        pl.BlockSpec((D, 1), lambda p, i: (0, 0)),
        pl.BlockSpec((1, 1), lambda p, i: (0, 0)),
        pl.BlockSpec((R, 1), lambda p, i: (i, 0)),
        pl.BlockSpec((D, DO), lambda p, i: (0, 0)),
        pl.BlockSpec((1, DO), lambda p, i: (0, 0)),
    ],
    out_specs=[pl.BlockSpec((G, DO), lambda p, i: (0, 0))],
    out_shape=[jax.ShapeDtypeStruct((G, DO), jnp.float32)],
    scratch_shapes=[
        pltpu.VMEM((N, D), jnp.float32),
        pltpu.VMEM((N, 1), jnp.float32),
        pltpu.VMEM((1, G), jnp.float32),
        pltpu.VMEM((G, 1), jnp.float32),
        pltpu.VMEM((G, D), jnp.float32),
        pltpu.VMEM((8, D), jnp.float32),
    ],
)


def kernel(x, edge_index, batch, W1, b1, W2, b2, W3, b3,
           bn1_g, bn1_b, bn2_g, bn2_b, bn3_g, bn3_b,
           gate_W, gate_b, fc_W, fc_b):
    src = edge_index[0].reshape(NW, CH, K)
    dst = edge_index[1].reshape(NW, CH, K)
    batchc = batch.reshape(N, 1)

    deg2 = _deg_kernel(dst)
    deg0 = deg2[:N].reshape(N, 1)
    deg1 = deg2[NPAD:NPAD + N].reshape(N, 1)
    dinv, y = _prep_call(deg0, deg1, x, W1)

    acc = _edge_kernel(y, src, dst)
    y = _layer_call(acc, y, dinv, b1.reshape(1, D),
                    bn1_g.reshape(1, D), bn1_b.reshape(1, D), W2)[0]

    acc = _edge_kernel(y, src, dst)
    y = _layer_call(acc, y, dinv, b2.reshape(1, D),
                    bn2_g.reshape(1, D), bn2_b.reshape(1, D), W3)[0]

    acc = _edge_kernel(y, src, dst)
    out = _layer3_call(acc, y, dinv, b3.reshape(1, D),
                       bn3_g.reshape(1, D), bn3_b.reshape(1, D),
                       gate_W, gate_b.reshape(1, 1), batchc,
                       fc_W, fc_b.reshape(1, DO))[0]
    return out


# speedup vs baseline: 1.2740x; 1.0353x over previous
import functools

import jax
import jax.numpy as jnp
from jax import lax
from jax.experimental import pallas as pl
from jax.experimental.pallas import tpu as pltpu
from jax.experimental.pallas import tpu_sc as plsc

N = 10000
E = 320000
D = 128
G = 64
DO = 16
EPS_BN = 1e-5

NC = 2
NS = 16
NW = NC * NS
K = 128
CH = 80
PADE = NW * CH * K - E
NPAD = 10240
ROWS_PT = NPAD // NS
DPT = NPAD // NS
ZR = 64
SUP = 40
NSB = CH // SUP

R = 2000
NB = N // R

_mesh = plsc.VectorSubcoreMesh(core_axis_name="c", subcore_axis_name="s")


@functools.partial(
    pl.kernel,
    out_type=jax.ShapeDtypeStruct((NC * NPAD,), jnp.float32),
    mesh=_mesh,
    scratch_types=[
        pltpu.VMEM_SHARED((NPAD,), jnp.float32),
        pltpu.VMEM((CH, K), jnp.int32),
        pltpu.VMEM((K,), jnp.float32),
        pltpu.VMEM((DPT,), jnp.float32),
    ],
)
def _deg_kernel(dst_hbm, out_hbm, deg_sp, idx_v, ones_v, z_v):
    cid = lax.axis_index("c")
    sid = lax.axis_index("s")
    wid = sid * NC + cid
    zero16 = jnp.zeros((16,), jnp.float32)
    one16 = jnp.ones((16,), jnp.float32)

    def _zb(i, c):
        z_v[pl.ds(i * 16, 16)] = zero16
        return c

    lax.fori_loop(0, DPT // 16, _zb, 0)
    for k in range(8):
        ones_v[pl.ds(min(k * 16, K - 16), 16)] = one16

    pltpu.sync_copy(z_v, deg_sp.at[pl.ds(sid * DPT, DPT)])
    plsc.subcore_barrier()

    pltpu.sync_copy(dst_hbm.at[wid], idx_v)

    def _body(j, c):
        pltpu.sync_copy(ones_v, deg_sp.at[idx_v.at[j]], add=True)
        return c

    lax.fori_loop(0, CH, _body, 0)
    plsc.subcore_barrier()
    pltpu.sync_copy(deg_sp.at[pl.ds(sid * DPT, DPT)],
                    out_hbm.at[pl.ds(cid * NPAD + sid * DPT, DPT)])


@functools.partial(
    pl.kernel,
    out_type=jax.ShapeDtypeStruct((NC, NPAD, D), jnp.float32),
    mesh=_mesh,
    scratch_types=[
        pltpu.VMEM_SHARED((NPAD, D), jnp.float32),
        pltpu.VMEM((SUP, K), jnp.int32),
        pltpu.VMEM((SUP, K), jnp.int32),
        pltpu.VMEM((2, K, D), jnp.float32),
        pltpu.SemaphoreType.DMA,
        pltpu.SemaphoreType.DMA,
    ],
)
def _edge_kernel(y_hbm, src_hbm, dst_hbm, out_hbm,
                 acc_sp, src_v, dst_v, rows_v, sem0, sem1):
    cid = lax.axis_index("c")
    sid = lax.axis_index("s")
    wid = sid * NC + cid
    zero16 = jnp.zeros((16,), jnp.float32)

    def _zrow(i, c):
        for k in range(D // 16):
            rows_v[0, i, pl.ds(k * 16, 16)] = zero16
        return c

    lax.fori_loop(0, ZR, _zrow, 0)
    base = sid * ROWS_PT
    for t in range(ROWS_PT // ZR):
        pltpu.sync_copy(rows_v.at[0, pl.ds(0, ZR)],
                        acc_sp.at[pl.ds(base + t * ZR, ZR)])
    plsc.subcore_barrier()

    def _sb_body(sb, c):
        pltpu.sync_copy(src_hbm.at[wid, pl.ds(sb * SUP, SUP)], src_v)
        pltpu.sync_copy(dst_hbm.at[wid, pl.ds(sb * SUP, SUP)], dst_v)
        pltpu.async_copy(y_hbm.at[src_v.at[0]], rows_v.at[0], sem0)

        def _body(g, c2):
            j = 2 * g
            cp1 = pltpu.async_copy(y_hbm.at[src_v.at[j + 1]], rows_v.at[1], sem1)
            pltpu.make_async_copy(y_hbm.at[src_v.at[j]], rows_v.at[0], sem0).wait()
            pltpu.sync_copy(rows_v.at[0], acc_sp.at[dst_v.at[j]], add=True)

            @pl.when(j + 2 < SUP)
            def _():
                pltpu.async_copy(y_hbm.at[src_v.at[j + 2]], rows_v.at[0], sem0)

            cp1.wait()
            pltpu.sync_copy(rows_v.at[1], acc_sp.at[dst_v.at[j + 1]], add=True)
            return c2

        lax.fori_loop(0, SUP // 2, _body, 0)
        return c

    lax.fori_loop(0, NSB, _sb_body, 0)
    plsc.subcore_barrier()
    pltpu.sync_copy(acc_sp.at[pl.ds(base, ROWS_PT)],
                    out_hbm.at[cid, pl.ds(base, ROWS_PT)])


def _prep_body(deg0_ref, deg1_ref, x_ref, w_ref, dinv_ref, y_ref):
    dinv = lax.rsqrt(deg0_ref[...] + deg1_ref[...] + 1.0)
    dinv_ref[...] = dinv
    y_ref[...] = jnp.dot(x_ref[...], w_ref[...],
                         preferred_element_type=jnp.float32) * dinv


_prep_call = pl.pallas_call(
    _prep_body,
    grid=(NB,),
    in_specs=[
        pl.BlockSpec((R, 1), lambda i: (i, 0)),
        pl.BlockSpec((R, 1), lambda i: (i, 0)),
        pl.BlockSpec((R, D), lambda i: (i, 0)),
        pl.BlockSpec((D, D), lambda i: (0, 0)),
    ],
    out_specs=[
        pl.BlockSpec((R, 1), lambda i: (i, 0)),
        pl.BlockSpec((R, D), lambda i: (i, 0)),
    ],
    out_shape=[
        jax.ShapeDtypeStruct((N, 1), jnp.float32),
        jax.ShapeDtypeStruct((NPAD, D), jnp.float32),
    ],
)


def _bn_h(z, stats_ref, g_ref, bb_ref):
    mu = stats_ref[0:1] * (1.0 / N)
    ms = stats_ref[1:2] * (1.0 / N)
    var = ms - mu * mu
    inv = lax.rsqrt(var + EPS_BN)
    return jnp.maximum((z - mu) * inv * g_ref[...] + bb_ref[...], 0.0)


def _stats_update(stats_ref, z, i):
    s = jnp.sum(z, axis=0, keepdims=True)
    ss = jnp.sum(z * z, axis=0, keepdims=True)
    upd = jnp.concatenate([s, ss, jnp.zeros((6, D), jnp.float32)], axis=0)

    @pl.when(i == 0)
    def _():
        stats_ref[...] = jnp.zeros((8, D), jnp.float32)

    stats_ref[...] += upd


def _layer_body(acc_ref, y_ref, dinv_ref, b_ref, g_ref, bb_ref, w_ref,
                out_ref, zbuf, stats_ref):
    p = pl.program_id(0)
    i = pl.program_id(1)

    @pl.when(p == 0)
    def _():
        z = (acc_ref[0] + acc_ref[1] + y_ref[...]) * dinv_ref[...] + b_ref[...]
        zbuf[pl.ds(i * R, R), :] = z
        _stats_update(stats_ref, z, i)

    @pl.when(p == 1)
    def _():
        h = _bn_h(zbuf[pl.ds(i * R, R), :], stats_ref, g_ref, bb_ref)
        out_ref[...] = jnp.dot(h, w_ref[...],
                               preferred_element_type=jnp.float32) * dinv_ref[...]


_layer_call = pl.pallas_call(
    _layer_body,
    grid=(2, NB),
    in_specs=[
        pl.BlockSpec((2, R, D), lambda p, i: (0, i * (1 - p), 0)),
        pl.BlockSpec((R, D), lambda p, i: (i * (1 - p), 0)),
        pl.BlockSpec((R, 1), lambda p, i: (i, 0)),
        pl.BlockSpec((1, D), lambda p, i: (0, 0)),
        pl.BlockSpec((1, D), lambda p, i: (0, 0)),
        pl.BlockSpec((1, D), lambda p, i: (0, 0)),
        pl.BlockSpec((D, D), lambda p, i: (0, 0)),
    ],
    out_specs=[pl.BlockSpec((R, D), lambda p, i: (i * p, 0))],
    out_shape=[jax.ShapeDtypeStruct((NPAD, D), jnp.float32)],
    scratch_shapes=[
        pltpu.VMEM((N, D), jnp.float32),
        pltpu.VMEM((8, D), jnp.float32),
    ],
)


def _layer3_body(acc_ref, y_ref, dinv_ref, b_ref, g_ref, bb_ref,
                 gw_ref, gb_ref, batch_ref, fcw_ref, fcb_ref,
                 out_ref, zbuf, logit_s, m_s, s_acc, u_acc, stats_ref):
    p = pl.program_id(0)
    i = pl.program_id(1)

    @pl.when(p == 0)
    def _():
        z = (acc_ref[0] + acc_ref[1] + y_ref[...]) * dinv_ref[...] + b_ref[...]
        zbuf[pl.ds(i * R, R), :] = z
        _stats_update(stats_ref, z, i)

    @pl.when(p == 1)
    def _():
        h = _bn_h(zbuf[pl.ds(i * R, R), :], stats_ref, g_ref, bb_ref)
        zbuf[pl.ds(i * R, R), :] = h
        logit = jnp.dot(h, gw_ref[...],
                        preferred_element_type=jnp.float32) + gb_ref[...]
        logit_s[pl.ds(i * R, R), :] = logit
        seg = lax.broadcasted_iota(jnp.int32, (R, G), 1)
        mask = seg == batch_ref[...]
        part = jnp.max(jnp.where(mask, logit, -1e30), axis=0, keepdims=True)

        @pl.when(i == 0)
        def _():
            m_s[...] = jnp.full((1, G), -1e30, jnp.float32)

        m_s[...] = jnp.maximum(m_s[...], part)

    @pl.when(p == 2)
    def _():
        h = zbuf[pl.ds(i * R, R), :]
        logit = logit_s[pl.ds(i * R, R), :]
        seg = lax.broadcasted_iota(jnp.int32, (R, G), 1)
        mask = (seg == batch_ref[...]).astype(jnp.float32)
        mnode = lax.dot_general(mask, m_s[...], (((1,), (1,)), ((), ())),
                                preferred_element_type=jnp.float32)
        ex = jnp.exp(logit - mnode)
        spart = lax.dot_general(mask, ex, (((0,), (0,)), ((), ())),
                                preferred_element_type=jnp.float32)
        upart = lax.dot_general(mask, h * ex, (((0,), (0,)), ((), ())),
                                preferred_element_type=jnp.float32)

        @pl.when(i == 0)
        def _():
            s_acc[...] = jnp.zeros((G, 1), jnp.float32)
            u_acc[...] = jnp.zeros((G, D), jnp.float32)

        s_acc[...] += spart
        u_acc[...] += upart

        @pl.when(i == NB - 1)
        def _():
            pooled = u_acc[...] / (s_acc[...] + 1e-16)
            out_ref[...] = jnp.dot(pooled, fcw_ref[...],
                                   preferred_element_type=jnp.float32) + fcb_ref[...]


_layer3_call = pl.pallas_call(
    _layer3_body,
    grid=(3, NB),
    in_specs=[
        pl.BlockSpec((2, R, D), lambda p, i: (0, jnp.where(p == 0, i, 0), 0)),
        pl.BlockSpec((R, D), lambda p, i: (jnp.where(p == 0, i, 0), 0)),
        pl.BlockSpec((R, 1), lambda p, i: (i, 0)),
        pl.BlockSpec((1, D), lambda p, i: (0, 0)),
        pl.BlockSpec((1, D), lambda p, i: (0, 0)),
        pl.BlockSpec((1, D), lambda p, i: (0, 0)),
        pl.BlockSpec((D, 1), lambda p, i: (0, 0)),
        pl.BlockSpec((1, 1), lambda p, i: (0, 0)),
        pl.BlockSpec((R, 1), lambda p, i: (i, 0)),
        pl.BlockSpec((D, DO), lambda p, i: (0, 0)),
        pl.BlockSpec((1, DO), lambda p, i: (0, 0)),
    ],
    out_specs=[pl.BlockSpec((G, DO), lambda p, i: (0, 0))],
    out_shape=[jax.ShapeDtypeStruct((G, DO), jnp.float32)],
    scratch_shapes=[
        pltpu.VMEM((N, D), jnp.float32),
        pltpu.VMEM((N, 1), jnp.float32),
        pltpu.VMEM((1, G), jnp.float32),
        pltpu.VMEM((G, 1), jnp.float32),
        pltpu.VMEM((G, D), jnp.float32),
        pltpu.VMEM((8, D), jnp.float32),
    ],
)


def kernel(x, edge_index, batch, W1, b1, W2, b2, W3, b3,
           bn1_g, bn1_b, bn2_g, bn2_b, bn3_g, bn3_b,
           gate_W, gate_b, fc_W, fc_b):
    pad_idx = N + (jnp.arange(PADE, dtype=jnp.int32) % (NPAD - N))
    src = jnp.concatenate([edge_index[0], pad_idx]).reshape(NW, CH, K)
    dst = jnp.concatenate([edge_index[1], pad_idx]).reshape(NW, CH, K)
    batchc = batch.reshape(N, 1)

    deg2 = _deg_kernel(dst)
    deg0 = deg2[:N].reshape(N, 1)
    deg1 = deg2[NPAD:NPAD + N].reshape(N, 1)
    dinv, y = _prep_call(deg0, deg1, x, W1)

    acc = _edge_kernel(y, src, dst)
    y = _layer_call(acc, y, dinv, b1.reshape(1, D),
                    bn1_g.reshape(1, D), bn1_b.reshape(1, D), W2)[0]

    acc = _edge_kernel(y, src, dst)
    y = _layer_call(acc, y, dinv, b2.reshape(1, D),
                    bn2_g.reshape(1, D), bn2_b.reshape(1, D), W3)[0]

    acc = _edge_kernel(y, src, dst)
    out = _layer3_call(acc, y, dinv, b3.reshape(1, D),
                       bn3_g.reshape(1, D), bn3_b.reshape(1, D),
                       gate_W, gate_b.reshape(1, 1), batchc,
                       fc_W, fc_b.reshape(1, DO))[0]
    return out


# speedup vs baseline: 1.2905x; 1.0129x over previous
import functools

import jax
import jax.numpy as jnp
from jax import lax
from jax.experimental import pallas as pl
from jax.experimental.pallas import tpu as pltpu
from jax.experimental.pallas import tpu_sc as plsc

N = 10000
E = 320000
D = 128
G = 64
DO = 16
EPS_BN = 1e-5

NC = 2
NS = 16
NW = NC * NS
K = 128
CH = 80
PADE = NW * CH * K - E
NPAD = 10240
ROWS_PT = NPAD // NS
DPT = NPAD // NS
ZR = 64
SUP = 40
NSB = CH // SUP

R = 5000
NB = N // R

_mesh = plsc.VectorSubcoreMesh(core_axis_name="c", subcore_axis_name="s")


@functools.partial(
    pl.kernel,
    out_type=jax.ShapeDtypeStruct((NC * NPAD,), jnp.float32),
    mesh=_mesh,
    scratch_types=[
        pltpu.VMEM_SHARED((NPAD,), jnp.float32),
        pltpu.VMEM((CH, K), jnp.int32),
        pltpu.VMEM((K,), jnp.float32),
        pltpu.VMEM((DPT,), jnp.float32),
    ],
)
def _deg_kernel(dst_hbm, out_hbm, deg_sp, idx_v, ones_v, z_v):
    cid = lax.axis_index("c")
    sid = lax.axis_index("s")
    wid = sid * NC + cid
    zero16 = jnp.zeros((16,), jnp.float32)
    one16 = jnp.ones((16,), jnp.float32)

    def _zb(i, c):
        z_v[pl.ds(i * 16, 16)] = zero16
        return c

    lax.fori_loop(0, DPT // 16, _zb, 0)
    for k in range(8):
        ones_v[pl.ds(min(k * 16, K - 16), 16)] = one16

    pltpu.sync_copy(z_v, deg_sp.at[pl.ds(sid * DPT, DPT)])
    plsc.subcore_barrier()

    pltpu.sync_copy(dst_hbm.at[wid], idx_v)

    def _body(j, c):
        pltpu.sync_copy(ones_v, deg_sp.at[idx_v.at[j]], add=True)
        return c

    lax.fori_loop(0, CH, _body, 0)
    plsc.subcore_barrier()
    pltpu.sync_copy(deg_sp.at[pl.ds(sid * DPT, DPT)],
                    out_hbm.at[pl.ds(cid * NPAD + sid * DPT, DPT)])


@functools.partial(
    pl.kernel,
    out_type=jax.ShapeDtypeStruct((NC, NPAD, D), jnp.float32),
    mesh=_mesh,
    scratch_types=[
        pltpu.VMEM_SHARED((NPAD, D), jnp.float32),
        pltpu.VMEM((SUP, K), jnp.int32),
        pltpu.VMEM((SUP, K), jnp.int32),
        pltpu.VMEM((2, K, D), jnp.float32),
        pltpu.SemaphoreType.DMA,
        pltpu.SemaphoreType.DMA,
    ],
)
def _edge_kernel(y_hbm, src_hbm, dst_hbm, out_hbm,
                 acc_sp, src_v, dst_v, rows_v, sem0, sem1):
    cid = lax.axis_index("c")
    sid = lax.axis_index("s")
    wid = sid * NC + cid
    zero16 = jnp.zeros((16,), jnp.float32)

    def _zrow(i, c):
        for k in range(D // 16):
            rows_v[0, i, pl.ds(k * 16, 16)] = zero16
        return c

    lax.fori_loop(0, ZR, _zrow, 0)
    base = sid * ROWS_PT
    for t in range(ROWS_PT // ZR):
        pltpu.sync_copy(rows_v.at[0, pl.ds(0, ZR)],
                        acc_sp.at[pl.ds(base + t * ZR, ZR)])
    plsc.subcore_barrier()

    def _sb_body(sb, c):
        pltpu.sync_copy(src_hbm.at[wid, pl.ds(sb * SUP, SUP)], src_v)
        pltpu.sync_copy(dst_hbm.at[wid, pl.ds(sb * SUP, SUP)], dst_v)
        pltpu.async_copy(y_hbm.at[src_v.at[0]], rows_v.at[0], sem0)

        def _body(g, c2):
            j = 2 * g
            cp1 = pltpu.async_copy(y_hbm.at[src_v.at[j + 1]], rows_v.at[1], sem1)
            pltpu.make_async_copy(y_hbm.at[src_v.at[j]], rows_v.at[0], sem0).wait()
            pltpu.sync_copy(rows_v.at[0], acc_sp.at[dst_v.at[j]], add=True)

            @pl.when(j + 2 < SUP)
            def _():
                pltpu.async_copy(y_hbm.at[src_v.at[j + 2]], rows_v.at[0], sem0)

            cp1.wait()
            pltpu.sync_copy(rows_v.at[1], acc_sp.at[dst_v.at[j + 1]], add=True)
            return c2

        lax.fori_loop(0, SUP // 2, _body, 0)
        return c

    lax.fori_loop(0, NSB, _sb_body, 0)
    plsc.subcore_barrier()
    pltpu.sync_copy(acc_sp.at[pl.ds(base, ROWS_PT)],
                    out_hbm.at[cid, pl.ds(base, ROWS_PT)])


def _prep_body(deg0_ref, deg1_ref, x_ref, w_ref, dinv_ref, y_ref):
    dinv = lax.rsqrt(deg0_ref[...] + deg1_ref[...] + 1.0)
    dinv_ref[...] = dinv
    y_ref[...] = jnp.dot(x_ref[...], w_ref[...],
                         preferred_element_type=jnp.float32) * dinv


_prep_call = pl.pallas_call(
    _prep_body,
    grid=(NB,),
    in_specs=[
        pl.BlockSpec((R, 1), lambda i: (i, 0)),
        pl.BlockSpec((R, 1), lambda i: (i, 0)),
        pl.BlockSpec((R, D), lambda i: (i, 0)),
        pl.BlockSpec((D, D), lambda i: (0, 0)),
    ],
    out_specs=[
        pl.BlockSpec((R, 1), lambda i: (i, 0)),
        pl.BlockSpec((R, D), lambda i: (i, 0)),
    ],
    out_shape=[
        jax.ShapeDtypeStruct((N, 1), jnp.float32),
        jax.ShapeDtypeStruct((NPAD, D), jnp.float32),
    ],
)


def _bn_h(z, stats_ref, g_ref, bb_ref):
    mu = stats_ref[0:1] * (1.0 / N)
    ms = stats_ref[1:2] * (1.0 / N)
    var = ms - mu * mu
    inv = lax.rsqrt(var + EPS_BN)
    return jnp.maximum((z - mu) * inv * g_ref[...] + bb_ref[...], 0.0)


def _stats_update(stats_ref, z, i):
    s = jnp.sum(z, axis=0, keepdims=True)
    ss = jnp.sum(z * z, axis=0, keepdims=True)
    upd = jnp.concatenate([s, ss, jnp.zeros((6, D), jnp.float32)], axis=0)

    @pl.when(i == 0)
    def _():
        stats_ref[...] = jnp.zeros((8, D), jnp.float32)

    stats_ref[...] += upd


def _layer_body(acc_ref, y_ref, dinv_ref, b_ref, g_ref, bb_ref, w_ref,
                out_ref, zbuf, stats_ref):
    p = pl.program_id(0)
    i = pl.program_id(1)

    @pl.when(p == 0)
    def _():
        z = (acc_ref[0] + acc_ref[1] + y_ref[...]) * dinv_ref[...] + b_ref[...]
        zbuf[pl.ds(i * R, R), :] = z
        _stats_update(stats_ref, z, i)

    @pl.when(p == 1)
    def _():
        h = _bn_h(zbuf[pl.ds(i * R, R), :], stats_ref, g_ref, bb_ref)
        out_ref[...] = jnp.dot(h, w_ref[...],
                               preferred_element_type=jnp.float32) * dinv_ref[...]


_layer_call = pl.pallas_call(
    _layer_body,
    grid=(2, NB),
    in_specs=[
        pl.BlockSpec((2, R, D), lambda p, i: (0, i * (1 - p), 0)),
        pl.BlockSpec((R, D), lambda p, i: (i * (1 - p), 0)),
        pl.BlockSpec((R, 1), lambda p, i: (i, 0)),
        pl.BlockSpec((1, D), lambda p, i: (0, 0)),
        pl.BlockSpec((1, D), lambda p, i: (0, 0)),
        pl.BlockSpec((1, D), lambda p, i: (0, 0)),
        pl.BlockSpec((D, D), lambda p, i: (0, 0)),
    ],
    out_specs=[pl.BlockSpec((R, D), lambda p, i: (i * p, 0))],
    out_shape=[jax.ShapeDtypeStruct((NPAD, D), jnp.float32)],
    scratch_shapes=[
        pltpu.VMEM((N, D), jnp.float32),
        pltpu.VMEM((8, D), jnp.float32),
    ],
)


def _layer3_body(acc_ref, y_ref, dinv_ref, b_ref, g_ref, bb_ref,
                 gw_ref, gb_ref, batch_ref, fcw_ref, fcb_ref,
                 out_ref, zbuf, logit_s, m_s, s_acc, u_acc, stats_ref):
    p = pl.program_id(0)
    i = pl.program_id(1)

    @pl.when(p == 0)
    def _():
        z = (acc_ref[0] + acc_ref[1] + y_ref[...]) * dinv_ref[...] + b_ref[...]
        zbuf[pl.ds(i * R, R), :] = z
        _stats_update(stats_ref, z, i)

    @pl.when(p == 1)
    def _():
        h = _bn_h(zbuf[pl.ds(i * R, R), :], stats_ref, g_ref, bb_ref)
        zbuf[pl.ds(i * R, R), :] = h
        logit = jnp.dot(h, gw_ref[...],
                        preferred_element_type=jnp.float32) + gb_ref[...]
        logit_s[pl.ds(i * R, R), :] = logit
        seg = lax.broadcasted_iota(jnp.int32, (R, G), 1)
        mask = seg == batch_ref[...]
        part = jnp.max(jnp.where(mask, logit, -1e30), axis=0, keepdims=True)

        @pl.when(i == 0)
        def _():
            m_s[...] = jnp.full((1, G), -1e30, jnp.float32)

        m_s[...] = jnp.maximum(m_s[...], part)

    @pl.when(p == 2)
    def _():
        h = zbuf[pl.ds(i * R, R), :]
        logit = logit_s[pl.ds(i * R, R), :]
        seg = lax.broadcasted_iota(jnp.int32, (R, G), 1)
        mask = (seg == batch_ref[...]).astype(jnp.float32)
        mnode = lax.dot_general(mask, m_s[...], (((1,), (1,)), ((), ())),
                                preferred_element_type=jnp.float32)
        ex = jnp.exp(logit - mnode)
        spart = lax.dot_general(mask, ex, (((0,), (0,)), ((), ())),
                                preferred_element_type=jnp.float32)
        upart = lax.dot_general(mask, h * ex, (((0,), (0,)), ((), ())),
                                preferred_element_type=jnp.float32)

        @pl.when(i == 0)
        def _():
            s_acc[...] = jnp.zeros((G, 1), jnp.float32)
            u_acc[...] = jnp.zeros((G, D), jnp.float32)

        s_acc[...] += spart
        u_acc[...] += upart

        @pl.when(i == NB - 1)
        def _():
            pooled = u_acc[...] / (s_acc[...] + 1e-16)
            out_ref[...] = jnp.dot(pooled, fcw_ref[...],
                                   preferred_element_type=jnp.float32) + fcb_ref[...]


_layer3_call = pl.pallas_call(
    _layer3_body,
    grid=(3, NB),
    in_specs=[
        pl.BlockSpec((2, R, D), lambda p, i: (0, jnp.where(p == 0, i, 0), 0)),
        pl.BlockSpec((R, D), lambda p, i: (jnp.where(p == 0, i, 0), 0)),
        pl.BlockSpec((R, 1), lambda p, i: (i, 0)),
        pl.BlockSpec((1, D), lambda p, i: (0, 0)),
        pl.BlockSpec((1, D), lambda p, i: (0, 0)),
        pl.BlockSpec((1, D), lambda p, i: (0, 0)),
        pl.BlockSpec((D, 1), lambda p, i: (0, 0)),
        pl.BlockSpec((1, 1), lambda p, i: (0, 0)),
        pl.BlockSpec((R, 1), lambda p, i: (i, 0)),
        pl.BlockSpec((D, DO), lambda p, i: (0, 0)),
        pl.BlockSpec((1, DO), lambda p, i: (0, 0)),
    ],
    out_specs=[pl.BlockSpec((G, DO), lambda p, i: (0, 0))],
    out_shape=[jax.ShapeDtypeStruct((G, DO), jnp.float32)],
    scratch_shapes=[
        pltpu.VMEM((N, D), jnp.float32),
        pltpu.VMEM((N, 1), jnp.float32),
        pltpu.VMEM((1, G), jnp.float32),
        pltpu.VMEM((G, 1), jnp.float32),
        pltpu.VMEM((G, D), jnp.float32),
        pltpu.VMEM((8, D), jnp.float32),
    ],
)


def kernel(x, edge_index, batch, W1, b1, W2, b2, W3, b3,
           bn1_g, bn1_b, bn2_g, bn2_b, bn3_g, bn3_b,
           gate_W, gate_b, fc_W, fc_b):
    pad_idx = N + (jnp.arange(PADE, dtype=jnp.int32) % (NPAD - N))
    src = jnp.concatenate([edge_index[0], pad_idx]).reshape(NW, CH, K)
    dst = jnp.concatenate([edge_index[1], pad_idx]).reshape(NW, CH, K)
    batchc = batch.reshape(N, 1)

    deg2 = _deg_kernel(dst)
    deg0 = deg2[:N].reshape(N, 1)
    deg1 = deg2[NPAD:NPAD + N].reshape(N, 1)
    dinv, y = _prep_call(deg0, deg1, x, W1)

    acc = _edge_kernel(y, src, dst)
    y = _layer_call(acc, y, dinv, b1.reshape(1, D),
                    bn1_g.reshape(1, D), bn1_b.reshape(1, D), W2)[0]

    acc = _edge_kernel(y, src, dst)
    y = _layer_call(acc, y, dinv, b2.reshape(1, D),
                    bn2_g.reshape(1, D), bn2_b.reshape(1, D), W3)[0]

    acc = _edge_kernel(y, src, dst)
    out = _layer3_call(acc, y, dinv, b3.reshape(1, D),
                       bn3_g.reshape(1, D), bn3_b.reshape(1, D),
                       gate_W, gate_b.reshape(1, 1), batchc,
                       fc_W, fc_b.reshape(1, DO))[0]
    return out


# speedup vs baseline: 1.3283x; 1.0294x over previous
import functools

import jax
import jax.numpy as jnp
from jax import lax
from jax.experimental import pallas as pl
from jax.experimental.pallas import tpu as pltpu
from jax.experimental.pallas import tpu_sc as plsc

N = 10000
E = 320000
D = 128
G = 64
DO = 16
EPS_BN = 1e-5

NC = 2
NS = 16
NW = NC * NS
K = 128
CH = 80
PADE = NW * CH * K - E
NPAD = 10240
ROWS_PT = NPAD // NS
DPT = NPAD // NS
ZR = 64
SUP = 40
NSB = CH // SUP

R = 5000
NB = N // R

_mesh = plsc.VectorSubcoreMesh(core_axis_name="c", subcore_axis_name="s")


@functools.partial(
    pl.kernel,
    out_type=jax.ShapeDtypeStruct((NC * NPAD,), jnp.float32),
    mesh=_mesh,
    scratch_types=[
        pltpu.VMEM_SHARED((NPAD,), jnp.float32),
        pltpu.VMEM((CH, K), jnp.int32),
        pltpu.VMEM((K,), jnp.float32),
        pltpu.VMEM((DPT,), jnp.float32),
    ],
)
def _deg_kernel(dst_hbm, out_hbm, deg_sp, idx_v, ones_v, z_v):
    cid = lax.axis_index("c")
    sid = lax.axis_index("s")
    wid = sid * NC + cid
    zero16 = jnp.zeros((16,), jnp.float32)
    one16 = jnp.ones((16,), jnp.float32)

    def _zb(i, c):
        z_v[pl.ds(i * 16, 16)] = zero16
        return c

    lax.fori_loop(0, DPT // 16, _zb, 0)
    for k in range(8):
        ones_v[pl.ds(min(k * 16, K - 16), 16)] = one16

    pltpu.sync_copy(z_v, deg_sp.at[pl.ds(sid * DPT, DPT)])
    plsc.subcore_barrier()

    pltpu.sync_copy(dst_hbm.at[wid], idx_v)

    def _body(j, c):
        pltpu.sync_copy(ones_v, deg_sp.at[idx_v.at[j]], add=True)
        return c

    lax.fori_loop(0, CH, _body, 0)
    plsc.subcore_barrier()
    pltpu.sync_copy(deg_sp.at[pl.ds(sid * DPT, DPT)],
                    out_hbm.at[pl.ds(cid * NPAD + sid * DPT, DPT)])


@functools.partial(
    pl.kernel,
    out_type=jax.ShapeDtypeStruct((NC, NPAD, D), jnp.float32),
    mesh=_mesh,
    scratch_types=[
        pltpu.VMEM_SHARED((NPAD, D), jnp.float32),
        pltpu.VMEM((SUP, K), jnp.int32),
        pltpu.VMEM((SUP, K), jnp.int32),
        pltpu.VMEM((2, K, D), jnp.float32),
        pltpu.SemaphoreType.DMA,
        pltpu.SemaphoreType.DMA,
    ],
)
def _edge_kernel(y_hbm, src_hbm, dst_hbm, out_hbm,
                 acc_sp, src_v, dst_v, rows_v, sem0, sem1):
    cid = lax.axis_index("c")
    sid = lax.axis_index("s")
    wid = sid * NC + cid
    zero16 = jnp.zeros((16,), jnp.float32)

    def _zrow(i, c):
        for k in range(D // 16):
            rows_v[0, i, pl.ds(k * 16, 16)] = zero16
        return c

    lax.fori_loop(0, ZR, _zrow, 0)
    base = sid * ROWS_PT
    for t in range(ROWS_PT // ZR):
        pltpu.sync_copy(rows_v.at[0, pl.ds(0, ZR)],
                        acc_sp.at[pl.ds(base + t * ZR, ZR)])
    plsc.subcore_barrier()

    def _sb_body(sb, c):
        pltpu.sync_copy(src_hbm.at[wid, pl.ds(sb * SUP, SUP)], src_v)
        pltpu.sync_copy(dst_hbm.at[wid, pl.ds(sb * SUP, SUP)], dst_v)
        pltpu.async_copy(y_hbm.at[src_v.at[0]], rows_v.at[0], sem0)

        def _body(g, c2):
            j = 2 * g
            cp1 = pltpu.async_copy(y_hbm.at[src_v.at[j + 1]], rows_v.at[1], sem1)
            pltpu.make_async_copy(y_hbm.at[src_v.at[j]], rows_v.at[0], sem0).wait()
            pltpu.sync_copy(rows_v.at[0], acc_sp.at[dst_v.at[j]], add=True)

            @pl.when(j + 2 < SUP)
            def _():
                pltpu.async_copy(y_hbm.at[src_v.at[j + 2]], rows_v.at[0], sem0)

            cp1.wait()
            pltpu.sync_copy(rows_v.at[1], acc_sp.at[dst_v.at[j + 1]], add=True)
            return c2

        lax.fori_loop(0, SUP // 2, _body, 0)
        return c

    lax.fori_loop(0, NSB, _sb_body, 0)
    plsc.subcore_barrier()
    pltpu.sync_copy(acc_sp.at[pl.ds(base, ROWS_PT)],
                    out_hbm.at[cid, pl.ds(base, ROWS_PT)])


def _prep_body(deg_ref, x_ref, w_ref, dinv_ref, y_ref):
    degsum = deg_ref[0] + deg_ref[1] + 1.0
    eye = (lax.broadcasted_iota(jnp.int32, (128, 128), 0)
           == lax.broadcasted_iota(jnp.int32, (128, 128), 1)).astype(jnp.float32)
    cols = [lax.dot_general(eye, degsum[r:r + 1, :], (((1,), (1,)), ((), ())),
                            preferred_element_type=jnp.float32)
            for r in range(NPAD // 128)]
    dinv = lax.rsqrt(jnp.concatenate(cols, axis=0)[:N])
    dinv_ref[...] = dinv
    y_ref[pl.ds(0, N), :] = jnp.dot(x_ref[...], w_ref[...],
                                    preferred_element_type=jnp.float32) * dinv


_prep_call = pl.pallas_call(
    _prep_body,
    grid=(1,),
    in_specs=[
        pl.BlockSpec((2, NPAD // 128, 128), lambda i: (0, 0, 0)),
        pl.BlockSpec((N, D), lambda i: (0, 0)),
        pl.BlockSpec((D, D), lambda i: (0, 0)),
    ],
    out_specs=[
        pl.BlockSpec((N, 1), lambda i: (0, 0)),
        pl.BlockSpec((NPAD, D), lambda i: (0, 0)),
    ],
    out_shape=[
        jax.ShapeDtypeStruct((N, 1), jnp.float32),
        jax.ShapeDtypeStruct((NPAD, D), jnp.float32),
    ],
)


def _bn_h(z, stats_ref, g_ref, bb_ref):
    mu = stats_ref[0:1] * (1.0 / N)
    ms = stats_ref[1:2] * (1.0 / N)
    var = ms - mu * mu
    inv = lax.rsqrt(var + EPS_BN)
    return jnp.maximum((z - mu) * inv * g_ref[...] + bb_ref[...], 0.0)


def _stats_update(stats_ref, z, i):
    s = jnp.sum(z, axis=0, keepdims=True)
    ss = jnp.sum(z * z, axis=0, keepdims=True)
    upd = jnp.concatenate([s, ss, jnp.zeros((6, D), jnp.float32)], axis=0)

    @pl.when(i == 0)
    def _():
        stats_ref[...] = jnp.zeros((8, D), jnp.float32)

    stats_ref[...] += upd


def _layer_body(acc_ref, y_ref, dinv_ref, b_ref, g_ref, bb_ref, w_ref,
                out_ref, zbuf, stats_ref):
    p = pl.program_id(0)
    i = pl.program_id(1)

    @pl.when(p == 0)
    def _():
        z = (acc_ref[0] + acc_ref[1] + y_ref[...]) * dinv_ref[...] + b_ref[...]
        zbuf[pl.ds(i * R, R), :] = z
        _stats_update(stats_ref, z, i)

    @pl.when(p == 1)
    def _():
        h = _bn_h(zbuf[pl.ds(i * R, R), :], stats_ref, g_ref, bb_ref)
        out_ref[...] = jnp.dot(h, w_ref[...],
                               preferred_element_type=jnp.float32) * dinv_ref[...]


_layer_call = pl.pallas_call(
    _layer_body,
    grid=(2, NB),
    in_specs=[
        pl.BlockSpec((2, R, D), lambda p, i: (0, i * (1 - p), 0)),
        pl.BlockSpec((R, D), lambda p, i: (i * (1 - p), 0)),
        pl.BlockSpec((R, 1), lambda p, i: (i, 0)),
        pl.BlockSpec((1, D), lambda p, i: (0, 0)),
        pl.BlockSpec((1, D), lambda p, i: (0, 0)),
        pl.BlockSpec((1, D), lambda p, i: (0, 0)),
        pl.BlockSpec((D, D), lambda p, i: (0, 0)),
    ],
    out_specs=[pl.BlockSpec((R, D), lambda p, i: (i * p, 0))],
    out_shape=[jax.ShapeDtypeStruct((NPAD, D), jnp.float32)],
    scratch_shapes=[
        pltpu.VMEM((N, D), jnp.float32),
        pltpu.VMEM((8, D), jnp.float32),
    ],
)


def _layer3_body(acc_ref, y_ref, dinv_ref, b_ref, g_ref, bb_ref,
                 gw_ref, gb_ref, batch_ref, fcw_ref, fcb_ref,
                 out_ref, zbuf, logit_s, m_s, s_acc, u_acc, stats_ref):
    p = pl.program_id(0)
    i = pl.program_id(1)

    @pl.when(p == 0)
    def _():
        z = (acc_ref[0] + acc_ref[1] + y_ref[...]) * dinv_ref[...] + b_ref[...]
        zbuf[pl.ds(i * R, R), :] = z
        _stats_update(stats_ref, z, i)

    @pl.when(p == 1)
    def _():
        h = _bn_h(zbuf[pl.ds(i * R, R), :], stats_ref, g_ref, bb_ref)
        zbuf[pl.ds(i * R, R), :] = h
        logit = jnp.dot(h, gw_ref[...],
                        preferred_element_type=jnp.float32) + gb_ref[...]
        logit_s[pl.ds(i * R, R), :] = logit
        seg = lax.broadcasted_iota(jnp.int32, (R, G), 1)
        mask = seg == batch_ref[...]
        part = jnp.max(jnp.where(mask, logit, -1e30), axis=0, keepdims=True)

        @pl.when(i == 0)
        def _():
            m_s[...] = jnp.full((1, G), -1e30, jnp.float32)

        m_s[...] = jnp.maximum(m_s[...], part)

    @pl.when(p == 2)
    def _():
        h = zbuf[pl.ds(i * R, R), :]
        logit = logit_s[pl.ds(i * R, R), :]
        seg = lax.broadcasted_iota(jnp.int32, (R, G), 1)
        mask = (seg == batch_ref[...]).astype(jnp.float32)
        mnode = lax.dot_general(mask, m_s[...], (((1,), (1,)), ((), ())),
                                preferred_element_type=jnp.float32)
        ex = jnp.exp(logit - mnode)
        spart = lax.dot_general(mask, ex, (((0,), (0,)), ((), ())),
                                preferred_element_type=jnp.float32)
        upart = lax.dot_general(mask, h * ex, (((0,), (0,)), ((), ())),
                                preferred_element_type=jnp.float32)

        @pl.when(i == 0)
        def _():
            s_acc[...] = jnp.zeros((G, 1), jnp.float32)
            u_acc[...] = jnp.zeros((G, D), jnp.float32)

        s_acc[...] += spart
        u_acc[...] += upart

        @pl.when(i == NB - 1)
        def _():
            pooled = u_acc[...] / (s_acc[...] + 1e-16)
            out_ref[...] = jnp.dot(pooled, fcw_ref[...],
                                   preferred_element_type=jnp.float32) + fcb_ref[...]


_layer3_call = pl.pallas_call(
    _layer3_body,
    grid=(3, NB),
    in_specs=[
        pl.BlockSpec((2, R, D), lambda p, i: (0, jnp.where(p == 0, i, 0), 0)),
        pl.BlockSpec((R, D), lambda p, i: (jnp.where(p == 0, i, 0), 0)),
        pl.BlockSpec((R, 1), lambda p, i: (i, 0)),
        pl.BlockSpec((1, D), lambda p, i: (0, 0)),
        pl.BlockSpec((1, D), lambda p, i: (0, 0)),
        pl.BlockSpec((1, D), lambda p, i: (0, 0)),
        pl.BlockSpec((D, 1), lambda p, i: (0, 0)),
        pl.BlockSpec((1, 1), lambda p, i: (0, 0)),
        pl.BlockSpec((R, 1), lambda p, i: (i, 0)),
        pl.BlockSpec((D, DO), lambda p, i: (0, 0)),
        pl.BlockSpec((1, DO), lambda p, i: (0, 0)),
    ],
    out_specs=[pl.BlockSpec((G, DO), lambda p, i: (0, 0))],
    out_shape=[jax.ShapeDtypeStruct((G, DO), jnp.float32)],
    scratch_shapes=[
        pltpu.VMEM((N, D), jnp.float32),
        pltpu.VMEM((N, 1), jnp.float32),
        pltpu.VMEM((1, G), jnp.float32),
        pltpu.VMEM((G, 1), jnp.float32),
        pltpu.VMEM((G, D), jnp.float32),
        pltpu.VMEM((8, D), jnp.float32),
    ],
)


def kernel(x, edge_index, batch, W1, b1, W2, b2, W3, b3,
           bn1_g, bn1_b, bn2_g, bn2_b, bn3_g, bn3_b,
           gate_W, gate_b, fc_W, fc_b):
    pad_idx = N + (jnp.arange(PADE, dtype=jnp.int32) % (NPAD - N))
    src = jnp.concatenate([edge_index[0], pad_idx]).reshape(NW, CH, K)
    dst = jnp.concatenate([edge_index[1], pad_idx]).reshape(NW, CH, K)
    batchc = batch.reshape(N, 1)

    deg2 = _deg_kernel(dst).reshape(NC, NPAD // 128, 128)
    dinv, y = _prep_call(deg2, x, W1)

    acc = _edge_kernel(y, src, dst)
    y = _layer_call(acc, y, dinv, b1.reshape(1, D),
                    bn1_g.reshape(1, D), bn1_b.reshape(1, D), W2)[0]

    acc = _edge_kernel(y, src, dst)
    y = _layer_call(acc, y, dinv, b2.reshape(1, D),
                    bn2_g.reshape(1, D), bn2_b.reshape(1, D), W3)[0]

    acc = _edge_kernel(y, src, dst)
    out = _layer3_call(acc, y, dinv, b3.reshape(1, D),
                       bn3_g.reshape(1, D), bn3_b.reshape(1, D),
                       gate_W, gate_b.reshape(1, 1), batchc,
                       fc_W, fc_b.reshape(1, DO))[0]
    return out


# speedup vs baseline: 1.3475x; 1.0144x over previous
import functools

import jax
import jax.numpy as jnp
from jax import lax
from jax.experimental import pallas as pl
from jax.experimental.pallas import tpu as pltpu
from jax.experimental.pallas import tpu_sc as plsc

N = 10000
E = 320000
D = 128
G = 64
DO = 16
EPS_BN = 1e-5

NC = 2
NS = 16
NW = NC * NS
K = 128
CH = 80
PADE = NW * CH * K - E
NPAD = 10240
ROWS_PT = NPAD // NS
DPT = NPAD // NS
ZR = 64
SUP = 40
NSB = CH // SUP

R = 5000
NB = N // R

_mesh = plsc.VectorSubcoreMesh(core_axis_name="c", subcore_axis_name="s")


@functools.partial(
    pl.kernel,
    out_type=jax.ShapeDtypeStruct((NC * NPAD,), jnp.float32),
    mesh=_mesh,
    scratch_types=[
        pltpu.VMEM_SHARED((NPAD,), jnp.float32),
        pltpu.VMEM((CH, K), jnp.int32),
        pltpu.VMEM((K,), jnp.float32),
        pltpu.VMEM((DPT,), jnp.float32),
    ],
)
def _deg_kernel(dst_hbm, out_hbm, deg_sp, idx_v, ones_v, z_v):
    cid = lax.axis_index("c")
    sid = lax.axis_index("s")
    wid = sid * NC + cid
    zero16 = jnp.zeros((16,), jnp.float32)
    one16 = jnp.ones((16,), jnp.float32)

    def _zb(i, c):
        z_v[pl.ds(i * 16, 16)] = zero16
        return c

    lax.fori_loop(0, DPT // 16, _zb, 0)
    for k in range(8):
        ones_v[pl.ds(min(k * 16, K - 16), 16)] = one16

    pltpu.sync_copy(z_v, deg_sp.at[pl.ds(sid * DPT, DPT)])
    plsc.subcore_barrier()

    pltpu.sync_copy(dst_hbm.at[wid], idx_v)

    def _body(j, c):
        pltpu.sync_copy(ones_v, deg_sp.at[idx_v.at[j]], add=True)
        return c

    lax.fori_loop(0, CH, _body, 0)
    plsc.subcore_barrier()
    pltpu.sync_copy(deg_sp.at[pl.ds(sid * DPT, DPT)],
                    out_hbm.at[pl.ds(cid * NPAD + sid * DPT, DPT)])


@functools.partial(
    pl.kernel,
    out_type=jax.ShapeDtypeStruct((NC, NPAD, D), jnp.float32),
    mesh=_mesh,
    scratch_types=[
        pltpu.VMEM_SHARED((NPAD, D), jnp.float32),
        pltpu.VMEM((SUP, K), jnp.int32),
        pltpu.VMEM((SUP, K), jnp.int32),
        pltpu.VMEM((2, K, D), jnp.float32),
        pltpu.SemaphoreType.DMA,
        pltpu.SemaphoreType.DMA,
    ],
)
def _edge_kernel(y_hbm, src_hbm, dst_hbm, out_hbm,
                 acc_sp, src_v, dst_v, rows_v, sem0, sem1):
    cid = lax.axis_index("c")
    sid = lax.axis_index("s")
    wid = sid * NC + cid
    zero16 = jnp.zeros((16,), jnp.float32)

    def _zrow(i, c):
        for k in range(D // 16):
            rows_v[0, i, pl.ds(k * 16, 16)] = zero16
        return c

    lax.fori_loop(0, ZR, _zrow, 0)
    base = sid * ROWS_PT
    for t in range(ROWS_PT // ZR):
        pltpu.sync_copy(rows_v.at[0, pl.ds(0, ZR)],
                        acc_sp.at[pl.ds(base + t * ZR, ZR)])
    plsc.subcore_barrier()

    def _sb_body(sb, c):
        pltpu.sync_copy(src_hbm.at[wid, pl.ds(sb * SUP, SUP)], src_v)
        pltpu.sync_copy(dst_hbm.at[wid, pl.ds(sb * SUP, SUP)], dst_v)
        pltpu.async_copy(y_hbm.at[src_v.at[0]], rows_v.at[0], sem0)

        def _body(g, c2):
            j = 2 * g
            cp1 = pltpu.async_copy(y_hbm.at[src_v.at[j + 1]], rows_v.at[1], sem1)
            pltpu.make_async_copy(y_hbm.at[src_v.at[j]], rows_v.at[0], sem0).wait()
            pltpu.sync_copy(rows_v.at[0], acc_sp.at[dst_v.at[j]], add=True)

            @pl.when(j + 2 < SUP)
            def _():
                pltpu.async_copy(y_hbm.at[src_v.at[j + 2]], rows_v.at[0], sem0)

            cp1.wait()
            pltpu.sync_copy(rows_v.at[1], acc_sp.at[dst_v.at[j + 1]], add=True)
            return c2

        lax.fori_loop(0, SUP // 2, _body, 0)
        return c

    lax.fori_loop(0, NSB, _sb_body, 0)
    plsc.subcore_barrier()
    pltpu.sync_copy(acc_sp.at[pl.ds(base, ROWS_PT)],
                    out_hbm.at[cid, pl.ds(base, ROWS_PT)])


def _col_from_rows(rows):
    eye = (lax.broadcasted_iota(jnp.int32, (128, 128), 0)
           == lax.broadcasted_iota(jnp.int32, (128, 128), 1)).astype(jnp.float32)
    cols = [lax.dot_general(eye, rows[r:r + 1, :], (((1,), (1,)), ((), ())),
                            preferred_element_type=jnp.float32)
            for r in range(NPAD // 128)]
    return jnp.concatenate(cols, axis=0)[:N]


def _prep_body(deg_ref, x_ref, w_ref, dinv2_ref, y_ref):
    dinv2 = lax.rsqrt(deg_ref[0] + deg_ref[1] + 1.0)
    dinv2_ref[...] = dinv2
    dinv = _col_from_rows(dinv2)
    y_ref[pl.ds(0, N), :] = jnp.dot(x_ref[...], w_ref[...],
                                    preferred_element_type=jnp.float32) * dinv


_prep_call = pl.pallas_call(
    _prep_body,
    grid=(1,),
    in_specs=[
        pl.BlockSpec((2, NPAD // 128, 128), lambda i: (0, 0, 0)),
        pl.BlockSpec((N, D), lambda i: (0, 0)),
        pl.BlockSpec((D, D), lambda i: (0, 0)),
    ],
    out_specs=[
        pl.BlockSpec((NPAD // 128, 128), lambda i: (0, 0)),
        pl.BlockSpec((NPAD, D), lambda i: (0, 0)),
    ],
    out_shape=[
        jax.ShapeDtypeStruct((NPAD // 128, 128), jnp.float32),
        jax.ShapeDtypeStruct((NPAD, D), jnp.float32),
    ],
)


def _bn_h(z, stats_ref, g_ref, bb_ref):
    mu = stats_ref[0:1] * (1.0 / N)
    ms = stats_ref[1:2] * (1.0 / N)
    var = ms - mu * mu
    inv = lax.rsqrt(var + EPS_BN)
    return jnp.maximum((z - mu) * inv * g_ref[...] + bb_ref[...], 0.0)


def _stats_update(stats_ref, z, i):
    s = jnp.sum(z, axis=0, keepdims=True)
    ss = jnp.sum(z * z, axis=0, keepdims=True)
    upd = jnp.concatenate([s, ss, jnp.zeros((6, D), jnp.float32)], axis=0)

    @pl.when(i == 0)
    def _():
        stats_ref[...] = jnp.zeros((8, D), jnp.float32)

    stats_ref[...] += upd


def _layer_body(acc_ref, y_ref, dinv2_ref, b_ref, g_ref, bb_ref, w_ref,
                out_ref, zbuf, stats_ref, dcol):
    p = pl.program_id(0)
    i = pl.program_id(1)

    @pl.when(jnp.logical_and(p == 0, i == 0))
    def _():
        dcol[...] = _col_from_rows(dinv2_ref[...])

    dv = dcol[pl.ds(i * R, R), :]

    @pl.when(p == 0)
    def _():
        z = (acc_ref[0] + acc_ref[1] + y_ref[...]) * dv + b_ref[...]
        zbuf[pl.ds(i * R, R), :] = z
        _stats_update(stats_ref, z, i)

    @pl.when(p == 1)
    def _():
        h = _bn_h(zbuf[pl.ds(i * R, R), :], stats_ref, g_ref, bb_ref)
        out_ref[...] = jnp.dot(h, w_ref[...],
                               preferred_element_type=jnp.float32) * dv


_layer_call = pl.pallas_call(
    _layer_body,
    grid=(2, NB),
    in_specs=[
        pl.BlockSpec((2, R, D), lambda p, i: (0, i * (1 - p), 0)),
        pl.BlockSpec((R, D), lambda p, i: (i * (1 - p), 0)),
        pl.BlockSpec((NPAD // 128, 128), lambda p, i: (0, 0)),
        pl.BlockSpec((1, D), lambda p, i: (0, 0)),
        pl.BlockSpec((1, D), lambda p, i: (0, 0)),
        pl.BlockSpec((1, D), lambda p, i: (0, 0)),
        pl.BlockSpec((D, D), lambda p, i: (0, 0)),
    ],
    out_specs=[pl.BlockSpec((R, D), lambda p, i: (i * p, 0))],
    out_shape=[jax.ShapeDtypeStruct((NPAD, D), jnp.float32)],
    scratch_shapes=[
        pltpu.VMEM((N, D), jnp.float32),
        pltpu.VMEM((8, D), jnp.float32),
        pltpu.VMEM((N, 1), jnp.float32),
    ],
)


def _layer3_body(acc_ref, y_ref, dinv2_ref, b_ref, g_ref, bb_ref,
                 gw_ref, gb_ref, batch_ref, fcw_ref, fcb_ref,
                 out_ref, zbuf, logit_s, m_s, s_acc, u_acc, stats_ref,
                 dcol, bcol):
    p = pl.program_id(0)
    i = pl.program_id(1)

    @pl.when(jnp.logical_and(p == 0, i == 0))
    def _():
        dcol[...] = _col_from_rows(dinv2_ref[...])
        bcol[...] = _col_from_rows(batch_ref[...])

    @pl.when(p == 0)
    def _():
        dv = dcol[pl.ds(i * R, R), :]
        z = (acc_ref[0] + acc_ref[1] + y_ref[...]) * dv + b_ref[...]
        zbuf[pl.ds(i * R, R), :] = z
        _stats_update(stats_ref, z, i)

    @pl.when(p == 1)
    def _():
        h = _bn_h(zbuf[pl.ds(i * R, R), :], stats_ref, g_ref, bb_ref)
        zbuf[pl.ds(i * R, R), :] = h
        logit = jnp.dot(h, gw_ref[...],
                        preferred_element_type=jnp.float32) + gb_ref[...]
        logit_s[pl.ds(i * R, R), :] = logit
        seg = lax.broadcasted_iota(jnp.int32, (R, G), 1).astype(jnp.float32)
        mask = seg == bcol[pl.ds(i * R, R), :]
        part = jnp.max(jnp.where(mask, logit, -1e30), axis=0, keepdims=True)

        @pl.when(i == 0)
        def _():
            m_s[...] = jnp.full((1, G), -1e30, jnp.float32)

        m_s[...] = jnp.maximum(m_s[...], part)

    @pl.when(p == 2)
    def _():
        h = zbuf[pl.ds(i * R, R), :]
        logit = logit_s[pl.ds(i * R, R), :]
        seg = lax.broadcasted_iota(jnp.int32, (R, G), 1).astype(jnp.float32)
        mask = (seg == bcol[pl.ds(i * R, R), :]).astype(jnp.float32)
        mnode = lax.dot_general(mask, m_s[...], (((1,), (1,)), ((), ())),
                                preferred_element_type=jnp.float32)
        ex = jnp.exp(logit - mnode)
        spart = lax.dot_general(mask, ex, (((0,), (0,)), ((), ())),
                                preferred_element_type=jnp.float32)
        upart = lax.dot_general(mask, h * ex, (((0,), (0,)), ((), ())),
                                preferred_element_type=jnp.float32)

        @pl.when(i == 0)
        def _():
            s_acc[...] = jnp.zeros((G, 1), jnp.float32)
            u_acc[...] = jnp.zeros((G, D), jnp.float32)

        s_acc[...] += spart
        u_acc[...] += upart

        @pl.when(i == NB - 1)
        def _():
            pooled = u_acc[...] / (s_acc[...] + 1e-16)
            out_ref[...] = jnp.dot(pooled, fcw_ref[...],
                                   preferred_element_type=jnp.float32) + fcb_ref[...]


_layer3_call = pl.pallas_call(
    _layer3_body,
    grid=(3, NB),
    in_specs=[
        pl.BlockSpec((2, R, D), lambda p, i: (0, jnp.where(p == 0, i, 0), 0)),
        pl.BlockSpec((R, D), lambda p, i: (jnp.where(p == 0, i, 0), 0)),
        pl.BlockSpec((NPAD // 128, 128), lambda p, i: (0, 0)),
        pl.BlockSpec((1, D), lambda p, i: (0, 0)),
        pl.BlockSpec((1, D), lambda p, i: (0, 0)),
        pl.BlockSpec((1, D), lambda p, i: (0, 0)),
        pl.BlockSpec((D, 1), lambda p, i: (0, 0)),
        pl.BlockSpec((1, 1), lambda p, i: (0, 0)),
        pl.BlockSpec((NPAD // 128, 128), lambda p, i: (0, 0)),
        pl.BlockSpec((D, DO), lambda p, i: (0, 0)),
        pl.BlockSpec((1, DO), lambda p, i: (0, 0)),
    ],
    out_specs=[pl.BlockSpec((G, DO), lambda p, i: (0, 0))],
    out_shape=[jax.ShapeDtypeStruct((G, DO), jnp.float32)],
    scratch_shapes=[
        pltpu.VMEM((N, D), jnp.float32),
        pltpu.VMEM((N, 1), jnp.float32),
        pltpu.VMEM((1, G), jnp.float32),
        pltpu.VMEM((G, 1), jnp.float32),
        pltpu.VMEM((G, D), jnp.float32),
        pltpu.VMEM((8, D), jnp.float32),
        pltpu.VMEM((N, 1), jnp.float32),
        pltpu.VMEM((N, 1), jnp.float32),
    ],
)


def kernel(x, edge_index, batch, W1, b1, W2, b2, W3, b3,
           bn1_g, bn1_b, bn2_g, bn2_b, bn3_g, bn3_b,
           gate_W, gate_b, fc_W, fc_b):
    pad_idx = N + (jnp.arange(PADE, dtype=jnp.int32) % (NPAD - N))
    src = jnp.concatenate([edge_index[0], pad_idx]).reshape(NW, CH, K)
    dst = jnp.concatenate([edge_index[1], pad_idx]).reshape(NW, CH, K)
    batch2 = jnp.concatenate(
        [batch, jnp.full((NPAD - N,), G, jnp.int32)]
    ).astype(jnp.float32).reshape(NPAD // 128, 128)

    deg2 = _deg_kernel(dst).reshape(NC, NPAD // 128, 128)
    dinv2, y = _prep_call(deg2, x, W1)

    acc = _edge_kernel(y, src, dst)
    y = _layer_call(acc, y, dinv2, b1.reshape(1, D),
                    bn1_g.reshape(1, D), bn1_b.reshape(1, D), W2)[0]

    acc = _edge_kernel(y, src, dst)
    y = _layer_call(acc, y, dinv2, b2.reshape(1, D),
                    bn2_g.reshape(1, D), bn2_b.reshape(1, D), W3)[0]

    acc = _edge_kernel(y, src, dst)
    out = _layer3_call(acc, y, dinv2, b3.reshape(1, D),
                       bn3_g.reshape(1, D), bn3_b.reshape(1, D),
                       gate_W, gate_b.reshape(1, 1), batch2,
                       fc_W, fc_b.reshape(1, DO))[0]
    return out
